# baseline (speedup 1.0000x reference)
import numpy as np
import jax
import jax.numpy as jnp
from jax.experimental import pallas as pl
from jax.experimental.pallas import tpu as pltpu

_PLACEHOLDER = 265
_B, _N, _D = 128, 77, 768
_T = 4 * _D
_INNER = 512
_KB = 512
_NK = _T // _KB
_HALF = _D // 2


def _emb_kernel(ts_ref, w1_ref, b1_ref, w2_ref, out_ref):
    k = pl.program_id(0)

    io = jax.lax.broadcasted_iota(jnp.int32, (1, _HALF), 1).astype(jnp.float32)
    freqs = jnp.exp(io * jnp.float32(-np.log(10000.0) / _HALF))
    args = ts_ref[...] * freqs
    t_emb = jnp.concatenate([jnp.cos(args), jnp.sin(args)], axis=-1)

    z1 = jnp.dot(t_emb, w1_ref[...], preferred_element_type=jnp.float32)
    z1 = z1 + b1_ref[...]
    z1 = z1 * jax.nn.sigmoid(z1)

    part = jnp.dot(z1, w2_ref[...], preferred_element_type=jnp.float32)

    @pl.when(k == 0)
    def _():
        out_ref[...] = part

    @pl.when(k > 0)
    def _():
        out_ref[...] = out_ref[...] + part


def _compute_emb(ts, time_W1, time_b1, time_W2):
    return pl.pallas_call(
        _emb_kernel,
        grid=(_NK,),
        in_specs=[
            pl.BlockSpec((_B, 1), lambda k: (0, 0)),
            pl.BlockSpec((_D, _KB), lambda k: (0, k)),
            pl.BlockSpec((1, _KB), lambda k: (0, k)),
            pl.BlockSpec((_KB, _T), lambda k: (k, 0)),
        ],
        out_specs=pl.BlockSpec((_B, _T), lambda k: (0, 0)),
        out_shape=jax.ShapeDtypeStruct((_B, _T), jnp.float32),
    )(ts, time_W1, time_b1.reshape(1, _T), time_W2)


def _tail_kernel(e_ref, b2_ref, embw_ref, embb_ref, wv_ref, wo_ref, bo_ref,
                 netw_ref, netb_ref, init_ref, tok_ref, pe_ref, col_ref):
    emb = e_ref[...] + b2_ref[...]
    s = emb * jax.nn.sigmoid(emb)
    h = jnp.dot(s, embw_ref[...], preferred_element_type=jnp.float32)
    h = h + embb_ref[...] + init_ref[...]
    v = jnp.dot(h, wv_ref[...], preferred_element_type=jnp.float32)
    x2 = jnp.dot(v, wo_ref[...], preferred_element_type=jnp.float32)
    x2 = x2 + bo_ref[...]
    pe = jnp.dot(x2, netw_ref[...], preferred_element_type=jnp.float32)
    pe_ref[...] = pe + netb_ref[...]
    io = jax.lax.broadcasted_iota(jnp.int32, (_B, _N), 1)
    col_ref[...] = jnp.max(
        jnp.where(tok_ref[...] == _PLACEHOLDER, io, 0), axis=1, keepdims=True)


def _compute_pe_cols(emb, time_b2, emb_W, emb_b, a2_Wv, a2_Wo, a2_bo,
                     net_W, net_b, init_emb, tok):
    full = lambda shape: pl.BlockSpec(shape, lambda: tuple(0 for _ in shape))
    return pl.pallas_call(
        _tail_kernel,
        in_specs=[
            full((_B, _T)), full((1, _T)), full((_T, _D)), full((1, _D)),
            full((_D, _INNER)), full((_INNER, _D)), full((1, _D)),
            full((_D, _D)), full((1, _D)), full((1, _D)), full((_B, _N)),
        ],
        out_specs=[full((_B, _D)), full((_B, 1))],
        out_shape=[jax.ShapeDtypeStruct((_B, _D), jnp.float32),
                   jax.ShapeDtypeStruct((_B, 1), jnp.int32)],
    )(emb, time_b2.reshape(1, _T), emb_W, emb_b.reshape(1, _D),
      a2_Wv, a2_Wo, a2_bo.reshape(1, _D), net_W, net_b.reshape(1, _D),
      init_emb, tok)


_RB = 8


def _assemble_kernel(tok_ref, pe_ref, emb_ref, out_ref):
    mask = tok_ref[...] == _PLACEHOLDER
    out_ref[...] = jnp.where(mask, pe_ref[...], emb_ref[...])


def _assemble(tok3, pe3, emb_text):
    return pl.pallas_call(
        _assemble_kernel,
        grid=(_B // _RB,),
        in_specs=[
            pl.BlockSpec((_RB, _N, 1), lambda i: (i, 0, 0)),
            pl.BlockSpec((_RB, 1, _D), lambda i: (i, 0, 0)),
            pl.BlockSpec((_RB, _N, _D), lambda i: (i, 0, 0)),
        ],
        out_specs=pl.BlockSpec((_RB, _N, _D), lambda i: (i, 0, 0)),
        out_shape=jax.ShapeDtypeStruct((_B, _N, _D), jnp.float32),
    )(tok3, pe3, emb_text)


def kernel(tokenized_text, embedded_text, timestep, time_W1, time_b1,
           time_W2, time_b2, emb_W, emb_b, a1_Wq, a1_Wk, a1_Wv, a1_Wo, a1_bo,
           a2_Wq, a2_Wk, a2_Wv, a2_Wo, a2_bo, net_W, net_b, init_emb):
    ts = timestep.astype(jnp.float32).reshape(_B, 1)
    emb = _compute_emb(ts, time_W1, time_b1, time_W2)
    pe, _cols = _compute_pe_cols(emb, time_b2, emb_W, emb_b, a2_Wv, a2_Wo,
                                 a2_bo, net_W, net_b, init_emb, tokenized_text)
    return _assemble(tokenized_text.reshape(_B, _N, 1),
                     pe.reshape(_B, 1, _D), embedded_text)


# speedup vs baseline: 1.9997x; 1.9997x over previous
import numpy as np
import jax
import jax.numpy as jnp
from jax.experimental import pallas as pl
from jax.experimental.pallas import tpu as pltpu

_PLACEHOLDER = 265
_B, _N, _D = 128, 77, 768
_T = 4 * _D
_INNER = 512
_KB = 512
_NK = _T // _KB
_HALF = _D // 2


def _emb_kernel(ts_ref, w1_ref, b1_ref, w2_ref, out_ref):
    k = pl.program_id(0)

    io = jax.lax.broadcasted_iota(jnp.int32, (1, _HALF), 1).astype(jnp.float32)
    freqs = jnp.exp(io * jnp.float32(-np.log(10000.0) / _HALF))
    args = ts_ref[...] * freqs
    t_emb = jnp.concatenate([jnp.cos(args), jnp.sin(args)], axis=-1)

    z1 = jnp.dot(t_emb, w1_ref[...], preferred_element_type=jnp.float32)
    z1 = z1 + b1_ref[...]
    z1 = z1 * jax.nn.sigmoid(z1)

    part = jnp.dot(z1, w2_ref[...], preferred_element_type=jnp.float32)

    @pl.when(k == 0)
    def _():
        out_ref[...] = part

    @pl.when(k > 0)
    def _():
        out_ref[...] = out_ref[...] + part


def _compute_emb(ts, time_W1, time_b1, time_W2):
    return pl.pallas_call(
        _emb_kernel,
        grid=(_NK,),
        in_specs=[
            pl.BlockSpec((_B, 1), lambda k: (0, 0)),
            pl.BlockSpec((_D, _KB), lambda k: (0, k)),
            pl.BlockSpec((1, _KB), lambda k: (0, k)),
            pl.BlockSpec((_KB, _T), lambda k: (k, 0)),
        ],
        out_specs=pl.BlockSpec((_B, _T), lambda k: (0, 0)),
        out_shape=jax.ShapeDtypeStruct((_B, _T), jnp.float32),
    )(ts, time_W1, time_b1.reshape(1, _T), time_W2)


def _tail_kernel(e_ref, b2_ref, embw_ref, embb_ref, wv_ref, wo_ref, bo_ref,
                 netw_ref, netb_ref, init_ref, tok_ref, pe_ref, col_ref):
    emb = e_ref[...] + b2_ref[...]
    s = emb * jax.nn.sigmoid(emb)
    h = jnp.dot(s, embw_ref[...], preferred_element_type=jnp.float32)
    h = h + embb_ref[...] + init_ref[...]
    v = jnp.dot(h, wv_ref[...], preferred_element_type=jnp.float32)
    x2 = jnp.dot(v, wo_ref[...], preferred_element_type=jnp.float32)
    x2 = x2 + bo_ref[...]
    pe = jnp.dot(x2, netw_ref[...], preferred_element_type=jnp.float32)
    pe_ref[...] = pe + netb_ref[...]
    io = jax.lax.broadcasted_iota(jnp.int32, (_B, _N), 1)
    col_ref[...] = jnp.max(
        jnp.where(tok_ref[...] == _PLACEHOLDER, io, 0), axis=1, keepdims=True)


def _compute_pe_cols(emb, time_b2, emb_W, emb_b, a2_Wv, a2_Wo, a2_bo,
                     net_W, net_b, init_emb, tok):
    full = lambda shape: pl.BlockSpec(shape, lambda: tuple(0 for _ in shape))
    return pl.pallas_call(
        _tail_kernel,
        in_specs=[
            full((_B, _T)), full((1, _T)), full((_T, _D)), full((1, _D)),
            full((_D, _INNER)), full((_INNER, _D)), full((1, _D)),
            full((_D, _D)), full((1, _D)), full((1, _D)), full((_B, _N)),
        ],
        out_specs=[full((_B, _D)), full((_B, 1))],
        out_shape=[jax.ShapeDtypeStruct((_B, _D), jnp.float32),
                   jax.ShapeDtypeStruct((_B, 1), jnp.int32)],
    )(emb, time_b2.reshape(1, _T), emb_W, emb_b.reshape(1, _D),
      a2_Wv, a2_Wo, a2_bo.reshape(1, _D), net_W, net_b.reshape(1, _D),
      init_emb, tok)


_NB = 7


def _assemble_kernel(tok_ref, pe_ref, emb_ref, out_ref):
    mask = tok_ref[...] == _PLACEHOLDER
    out_ref[...] = jnp.where(mask, pe_ref[...], emb_ref[...])


def _assemble_t(tok3, pe3, emb_t):
    return pl.pallas_call(
        _assemble_kernel,
        grid=(_N // _NB,),
        in_specs=[
            pl.BlockSpec((_NB, _B, 1), lambda i: (i, 0, 0)),
            pl.BlockSpec((1, _B, _D), lambda i: (0, 0, 0)),
            pl.BlockSpec((_NB, _B, _D), lambda i: (i, 0, 0)),
        ],
        out_specs=pl.BlockSpec((_NB, _B, _D), lambda i: (i, 0, 0)),
        out_shape=jax.ShapeDtypeStruct((_N, _B, _D), jnp.float32),
    )(tok3, pe3, emb_t)


def kernel(tokenized_text, embedded_text, timestep, time_W1, time_b1,
           time_W2, time_b2, emb_W, emb_b, a1_Wq, a1_Wk, a1_Wv, a1_Wo, a1_bo,
           a2_Wq, a2_Wk, a2_Wv, a2_Wo, a2_bo, net_W, net_b, init_emb):
    ts = timestep.astype(jnp.float32).reshape(_B, 1)
    emb = _compute_emb(ts, time_W1, time_b1, time_W2)
    pe, _cols = _compute_pe_cols(emb, time_b2, emb_W, emb_b, a2_Wv, a2_Wo,
                                 a2_bo, net_W, net_b, init_emb, tokenized_text)
    out_t = _assemble_t(tokenized_text.T.reshape(_N, _B, 1),
                        pe.reshape(1, _B, _D),
                        embedded_text.transpose(1, 0, 2))
    return out_t.transpose(1, 0, 2)
